# baseline (speedup 1.0000x reference)
import functools

import jax
import jax.numpy as jnp
from jax import lax
from jax.experimental import pallas as pl
from jax.experimental.pallas import tpu as pltpu
from jax.experimental.pallas import tpu_sc as plsc


def _build(N, B, dp, ds, NC, NS):
    NW = NC * NS
    n_per_w = N // NW
    s_per_w = B // NW
    C = 3200
    n_chunks = n_per_w // C

    mesh = plsc.VectorSubcoreMesh(core_axis_name="c", subcore_axis_name="s")

    @functools.partial(
        pl.kernel,
        mesh=mesh,
        compiler_params=pltpu.CompilerParams(use_tc_tiling_on_sc=False),
        out_type=(
            jax.ShapeDtypeStruct((N, dp), jnp.float32),
            jax.ShapeDtypeStruct((B, ds), jnp.float32),
        ),
        scratch_types=[
            pltpu.VMEM((C,), jnp.int32),
            pltpu.VMEM((C, dp), jnp.float32),
            pltpu.VMEM((s_per_w,), jnp.int32),
            pltpu.VMEM((s_per_w, ds), jnp.float32),
            pltpu.SemaphoreType.DMA,
        ],
    )
    def k(idx_hbm, spk_hbm, lutp_hbm, luts_hbm, outp_hbm, outs_hbm,
          idx_v, rows_v, sidx_v, srows_v, sem):
        wid = lax.axis_index("s") * NC + lax.axis_index("c")

        sbase = wid * s_per_w
        pltpu.sync_copy(spk_hbm.at[pl.ds(sbase, s_per_w)], sidx_v)
        pltpu.async_copy(luts_hbm.at[sidx_v], srows_v, sem).wait()
        pltpu.sync_copy(srows_v, outs_hbm.at[pl.ds(sbase, s_per_w)])

        base = wid * n_per_w

        def body(i, carry):
            off = base + i * C
            pltpu.sync_copy(idx_hbm.at[pl.ds(off, C)], idx_v)
            pltpu.async_copy(lutp_hbm.at[idx_v], rows_v, sem).wait()
            pltpu.sync_copy(rows_v, outp_hbm.at[pl.ds(off, C)])
            return carry

        lax.fori_loop(0, n_chunks, body, 0)

    return k


def kernel(sentence, speaker_id, Lut_P, Lut_S):
    B, L = sentence.shape
    dp = Lut_P.shape[1]
    ds = Lut_S.shape[1]
    N = B * L

    info = plsc.get_sparse_core_info()
    k = _build(N, B, dp, ds, info.num_cores, info.num_subcores)

    idx_flat = sentence.reshape(N).astype(jnp.int32)
    spk_flat = speaker_id.reshape(B).astype(jnp.int32)
    outp, outs = k(idx_flat, spk_flat, Lut_P, Lut_S)
    return outp.reshape(B, L, dp), outs


# speedup vs baseline: 1.0069x; 1.0069x over previous
import functools

import jax
import jax.numpy as jnp
from jax import lax
from jax.experimental import pallas as pl
from jax.experimental.pallas import tpu as pltpu
from jax.experimental.pallas import tpu_sc as plsc


def _build(N, B, dp, ds, NC, NS):
    NW = NC * NS
    n_per_w = N // NW
    s_per_w = B // NW
    C = 3200
    n_chunks = n_per_w // C
    n_pairs = n_chunks // 2

    mesh = plsc.VectorSubcoreMesh(core_axis_name="c", subcore_axis_name="s")

    @functools.partial(
        pl.kernel,
        mesh=mesh,
        compiler_params=pltpu.CompilerParams(use_tc_tiling_on_sc=False),
        out_type=(
            jax.ShapeDtypeStruct((N, dp), jnp.float32),
            jax.ShapeDtypeStruct((B, ds), jnp.float32),
        ),
        scratch_types=[
            pltpu.VMEM((C,), jnp.int32),
            pltpu.VMEM((C, dp), jnp.float32),
            pltpu.VMEM((C, dp), jnp.float32),
            pltpu.VMEM((s_per_w,), jnp.int32),
            pltpu.VMEM((s_per_w, ds), jnp.float32),
            pltpu.SemaphoreType.DMA,
            pltpu.SemaphoreType.DMA,
            pltpu.SemaphoreType.DMA,
        ],
    )
    def k(idx_hbm, spk_hbm, lutp_hbm, luts_hbm, outp_hbm, outs_hbm,
          idx_v, rows0_v, rows1_v, sidx_v, srows_v, sem_g, sem_s0, sem_s1):
        wid = lax.axis_index("s") * NC + lax.axis_index("c")
        rows_bufs = (rows0_v, rows1_v)
        store_sems = (sem_s0, sem_s1)

        sbase = wid * s_per_w
        pltpu.sync_copy(spk_hbm.at[pl.ds(sbase, s_per_w)], sidx_v)
        pltpu.async_copy(luts_hbm.at[sidx_v], srows_v, sem_g).wait()
        pltpu.sync_copy(srows_v, outs_hbm.at[pl.ds(sbase, s_per_w)])

        base = wid * n_per_w

        def body(j, carry):
            for b in range(2):
                i = j * 2 + b
                off = base + i * C
                rows_v, sem_s = rows_bufs[b], store_sems[b]
                pltpu.sync_copy(idx_hbm.at[pl.ds(off, C)], idx_v)

                @pl.when(j > 0)
                def _():
                    pltpu.make_async_copy(
                        rows_v, outp_hbm.at[pl.ds(base, C)], sem_s).wait()

                pltpu.async_copy(lutp_hbm.at[idx_v], rows_v, sem_g).wait()
                pltpu.async_copy(rows_v, outp_hbm.at[pl.ds(off, C)], sem_s)
            return carry

        lax.fori_loop(0, n_pairs, body, 0)
        for b in range(2):
            pltpu.make_async_copy(
                rows_bufs[b], outp_hbm.at[pl.ds(base, C)], store_sems[b]).wait()

    return k


def kernel(sentence, speaker_id, Lut_P, Lut_S):
    B, L = sentence.shape
    dp = Lut_P.shape[1]
    ds = Lut_S.shape[1]
    N = B * L

    info = plsc.get_sparse_core_info()
    k = _build(N, B, dp, ds, info.num_cores, info.num_subcores)

    idx_flat = sentence.reshape(N).astype(jnp.int32)
    spk_flat = speaker_id.reshape(B).astype(jnp.int32)
    outp, outs = k(idx_flat, spk_flat, Lut_P, Lut_S)
    return outp.reshape(B, L, dp), outs
